# baseline (speedup 1.0000x reference)
import functools

import jax
import jax.numpy as jnp
from jax import lax
from jax.experimental import pallas as pl
from jax.experimental.pallas import tpu as pltpu
from jax.experimental.pallas import tpu_sc as plsc

_NC, _NS, _LANES = 2, 16, 16

_K = 32
_G = 4
_GK = _G * _K
_N = 10000
_CO = 128
_CW = _CO // 2

_NPW0 = 464
_NPW1 = 176
_NG0 = _NPW0 // _G
_NG1 = _NPW1 // _G
_N_OUT = _NS * (_NPW0 + _NPW1)
_SPLIT_G = _NS * _NG0
_NG_TOT = _SPLIT_G + _NS * _NG1 + (_NG0 - _NG1)
_N_IDX = _NG_TOT * _G


def _pack_rows(v):
    vb = v.astype(jnp.bfloat16)
    lo = lax.bitcast_convert_type(vb[:, :_CW], jnp.int16).astype(jnp.int32)
    hi = lax.bitcast_convert_type(vb[:, _CW:], jnp.int16).astype(jnp.int32)
    return lax.bitwise_or(lax.bitwise_and(lo, jnp.int32(0xFFFF)),
                          lax.shift_left(hi, 16))


def _table_matmul_kernel(x_ref, w_ref, b_ref, a_ref, bt_ref):
    xb = x_ref[...]
    w = w_ref[...]
    c = xb.shape[0]
    w1 = w[:, :c]
    w2 = w[:, c:]
    dn = (((0,), (1,)), ((), ()))
    a = lax.dot_general(xb, w1 - w2, dn, preferred_element_type=jnp.float32)
    bt = lax.dot_general(xb, w2, dn,
                         preferred_element_type=jnp.float32) + b_ref[...]
    a_ref[...] = _pack_rows(a)
    bt_ref[...] = _pack_rows(bt)


def _build_tables(xs, w, b2d):
    return pl.pallas_call(
        _table_matmul_kernel,
        out_shape=[
            jax.ShapeDtypeStruct((_N, _CW), jnp.int32),
            jax.ShapeDtypeStruct((_N, _CW), jnp.int32),
        ],
    )(xs, w, b2d)


def _sc_body(i1_hbm, i0_hbm, a_hbm, bt_hbm, out_hbm,
             i1_v, i0_v, ra, rb, ov, ta_sh, sa0, sa1, sb0, sb1):
    sid = lax.axis_index("s")
    cid = lax.axis_index("c")

    rpt = _N // _NS

    @pl.when(cid == 0)
    def _stage():
        pltpu.sync_copy(a_hbm.at[pl.ds(sid * rpt, rpt)],
                        ta_sh.at[pl.ds(sid * rpt, rpt)])

    gb = jnp.where(cid == 0, sid * _NG0, _SPLIT_G + sid * _NG1)
    ng = jnp.where(cid == 0, _NG0, _NG1)

    pltpu.sync_copy(i1_hbm.at[pl.ds(gb, _NG0)], i1_v)
    pltpu.sync_copy(i0_hbm.at[pl.ds(gb, _NG0)], i0_v)

    plsc.subcore_barrier()

    sa = (sa0, sa1)
    sb = (sb0, sb1)

    def start(g, slot):
        @pl.when(cid == 0)
        def _a_spmem():
            pltpu.make_async_copy(ta_sh.at[i1_v.at[g]], ra.at[slot],
                                  sa[slot]).start()

        @pl.when(cid != 0)
        def _a_hbm():
            pltpu.make_async_copy(a_hbm.at[i1_v.at[g]], ra.at[slot],
                                  sa[slot]).start()

        pltpu.make_async_copy(bt_hbm.at[i0_v.at[g]], rb.at[slot], sb[slot]).start()

    def wait(slot):
        pltpu.make_async_copy(a_hbm.at[i1_v.at[0]], ra.at[slot], sa[slot]).wait()
        pltpu.make_async_copy(bt_hbm.at[i0_v.at[0]], rb.at[slot], sb[slot]).wait()

    start(0, 0)
    start(1, 1)

    nb = gb * _G

    @pl.loop(0, ng, step=2)
    def _outer(g):
        for slot in range(2):
            gg = g + slot
            wait(slot)
            himask = jnp.int32(-65536)

            def unpack2(word):
                lo = plsc.bitcast(lax.shift_left(word, 16), jnp.float32)
                hi = plsc.bitcast(lax.bitwise_and(word, himask), jnp.float32)
                return lo, hi

            for gi in range(_G):
                r0 = gi * _K

                def kbody(k, accs, _slot=slot, _r0=r0):
                    row = _r0 + k
                    new = []
                    for ci in range(4):
                        a0, a1 = unpack2(ra[_slot, row, pl.ds(ci * 16, 16)])
                        b0, b1 = unpack2(rb[_slot, row, pl.ds(ci * 16, 16)])
                        new.append(jnp.maximum(accs[2 * ci], a0 + b0))
                        new.append(jnp.maximum(accs[2 * ci + 1], a1 + b1))
                    return tuple(new)

                init = tuple(jnp.full((16,), -jnp.inf, jnp.float32)
                             for _ in range(8))
                accs = lax.fori_loop(0, _K, kbody, init)
                half = jnp.int32(0x8000)
                for ci in range(4):
                    lo = jnp.maximum(accs[2 * ci], 0.0)
                    hi = jnp.maximum(accs[2 * ci + 1], 0.0)
                    lo_i = lax.shift_right_logical(
                        plsc.bitcast(lo, jnp.int32) + half, 16)
                    hi_i = lax.bitwise_and(
                        plsc.bitcast(hi, jnp.int32) + half, himask)
                    ov[gi, pl.ds(ci * 16, 16)] = lax.bitwise_or(lo_i, hi_i)
            pltpu.sync_copy(ov, out_hbm.at[pl.ds(nb + gg * _G, _G)])

            @pl.when(gg + 2 < ng)
            def _refill():
                start(gg + 2, slot)


def _edge_reduce(i1, i0, a_tab, bt_tab):
    mesh = plsc.VectorSubcoreMesh(core_axis_name="c", subcore_axis_name="s",
                                  num_cores=_NC, num_subcores=_NS)
    f = pl.kernel(
        _sc_body,
        out_type=jax.ShapeDtypeStruct((_N_OUT, _CW), jnp.int32),
        mesh=mesh,
        compiler_params=pltpu.CompilerParams(needs_layout_passes=False,
                                             use_tc_tiling_on_sc=False),
        scratch_types=[
            pltpu.VMEM((_NG0, _GK), jnp.int32),
            pltpu.VMEM((_NG0, _GK), jnp.int32),
            pltpu.VMEM((2, _GK, _CW), jnp.int32),
            pltpu.VMEM((2, _GK, _CW), jnp.int32),
            pltpu.VMEM((_G, _CW), jnp.int32),
            pltpu.VMEM_SHARED((_N, _CW), jnp.int32),
            pltpu.SemaphoreType.DMA,
            pltpu.SemaphoreType.DMA,
            pltpu.SemaphoreType.DMA,
            pltpu.SemaphoreType.DMA,
        ],
    )
    return f(i1, i0, a_tab, bt_tab)


def kernel(x, edge_index, W, b):
    n = x.shape[2]
    xs = x[0, :, :, 0]
    idx = edge_index.reshape(2, n, _K).astype(jnp.int32)
    idx = jnp.pad(idx, ((0, 0), (0, _N_IDX - n), (0, 0)))
    idxg = idx.reshape(2, _NG_TOT, _GK)
    b2d = b.reshape(1, _CO)

    a_tab, bt_tab = _build_tables(xs, W, b2d)
    out = _edge_reduce(idxg[1], idxg[0], a_tab, bt_tab)
    v = lax.bitcast_convert_type(out[:n], jnp.bfloat16)
    res = v.transpose(2, 1, 0).reshape(_CO, n)
    return res.astype(jnp.float32)[None, :, :, None]


# speedup vs baseline: 1.1161x; 1.1161x over previous
import functools

import jax
import jax.numpy as jnp
from jax import lax
from jax.experimental import pallas as pl
from jax.experimental.pallas import tpu as pltpu
from jax.experimental.pallas import tpu_sc as plsc

_NC, _NS, _LANES = 2, 16, 16

_K = 32
_G = 4
_GK = _G * _K
_N = 10000
_CO = 128
_CW = _CO // 2

_NPW0 = 528
_NPW1 = 112
_NG0 = _NPW0 // _G
_NG1 = _NPW1 // _G
_N_OUT = _NS * (_NPW0 + _NPW1)
_SPLIT_G = _NS * _NG0
_NG_TOT = _SPLIT_G + _NS * _NG1
_N_IDX = _NG_TOT * _G


def _pack_rows(v):
    vb = v.astype(jnp.bfloat16)
    lo = lax.bitcast_convert_type(vb[:, :_CW], jnp.int16).astype(jnp.int32)
    hi = lax.bitcast_convert_type(vb[:, _CW:], jnp.int16).astype(jnp.int32)
    return lax.bitwise_or(lax.bitwise_and(lo, jnp.int32(0xFFFF)),
                          lax.shift_left(hi, 16))


def _table_matmul_kernel(x_ref, w_ref, b_ref, a_ref, bt_ref):
    xb = x_ref[...]
    w = w_ref[...]
    c = xb.shape[0]
    w1 = w[:, :c]
    w2 = w[:, c:]
    dn = (((0,), (1,)), ((), ()))
    a = lax.dot_general(xb, w1 - w2, dn, preferred_element_type=jnp.float32)
    bt = lax.dot_general(xb, w2, dn,
                         preferred_element_type=jnp.float32) + b_ref[...]
    a_ref[...] = _pack_rows(a)
    bt_ref[...] = _pack_rows(bt)


def _build_tables(xs, w, b2d):
    return pl.pallas_call(
        _table_matmul_kernel,
        out_shape=[
            jax.ShapeDtypeStruct((_N, _CW), jnp.int32),
            jax.ShapeDtypeStruct((_N, _CW), jnp.int32),
        ],
    )(xs, w, b2d)


def _sc_body(i1_hbm, i0_hbm, a_hbm, bt_hbm, out_hbm,
             i1_v, i0_v, ra, rb, ov, ta_sh, sa0, sa1, sb0, sb1):
    sid = lax.axis_index("s")
    cid = lax.axis_index("c")

    rpt = _N // _NS
    pltpu.sync_copy(a_hbm.at[pl.ds(sid * rpt, rpt)],
                    ta_sh.at[pl.ds(sid * rpt, rpt)])

    gb = jnp.where(cid == 0, sid * _NG0, _SPLIT_G + sid * _NG1)
    ng = jnp.where(cid == 0, _NG0, _NG1)

    @pl.when(cid == 0)
    def _stage_idx0():
        pltpu.sync_copy(i1_hbm.at[pl.ds(gb, _NG0)], i1_v)
        pltpu.sync_copy(i0_hbm.at[pl.ds(gb, _NG0)], i0_v)

    @pl.when(cid != 0)
    def _stage_idx1():
        pltpu.sync_copy(i1_hbm.at[pl.ds(gb, _NG1)], i1_v.at[pl.ds(0, _NG1)])
        pltpu.sync_copy(i0_hbm.at[pl.ds(gb, _NG1)], i0_v.at[pl.ds(0, _NG1)])

    plsc.subcore_barrier()

    sa = (sa0, sa1)
    sb = (sb0, sb1)

    def start(g, slot):
        pltpu.make_async_copy(ta_sh.at[i1_v.at[g]], ra.at[slot], sa[slot]).start()
        pltpu.make_async_copy(bt_hbm.at[i0_v.at[g]], rb.at[slot], sb[slot]).start()

    def wait(slot):
        pltpu.make_async_copy(a_hbm.at[i1_v.at[0]], ra.at[slot], sa[slot]).wait()
        pltpu.make_async_copy(bt_hbm.at[i0_v.at[0]], rb.at[slot], sb[slot]).wait()

    start(0, 0)
    start(1, 1)

    nb = gb * _G

    @pl.loop(0, ng, step=2)
    def _outer(g):
        for slot in range(2):
            gg = g + slot
            wait(slot)
            himask = jnp.int32(-65536)

            def unpack2(word):
                lo = plsc.bitcast(lax.shift_left(word, 16), jnp.float32)
                hi = plsc.bitcast(lax.bitwise_and(word, himask), jnp.float32)
                return lo, hi

            for gi in range(_G):
                r0 = gi * _K

                def kbody(k, accs, _slot=slot, _r0=r0):
                    row = _r0 + k
                    new = []
                    for ci in range(4):
                        a0, a1 = unpack2(ra[_slot, row, pl.ds(ci * 16, 16)])
                        b0, b1 = unpack2(rb[_slot, row, pl.ds(ci * 16, 16)])
                        new.append(jnp.maximum(accs[2 * ci], a0 + b0))
                        new.append(jnp.maximum(accs[2 * ci + 1], a1 + b1))
                    return tuple(new)

                init = tuple(jnp.full((16,), -jnp.inf, jnp.float32)
                             for _ in range(8))
                accs = lax.fori_loop(0, _K, kbody, init)
                half = jnp.int32(0x8000)
                for ci in range(4):
                    lo = jnp.maximum(accs[2 * ci], 0.0)
                    hi = jnp.maximum(accs[2 * ci + 1], 0.0)
                    lo_i = lax.shift_right_logical(
                        plsc.bitcast(lo, jnp.int32) + half, 16)
                    hi_i = lax.bitwise_and(
                        plsc.bitcast(hi, jnp.int32) + half, himask)
                    ov[gi, pl.ds(ci * 16, 16)] = lax.bitwise_or(lo_i, hi_i)
            pltpu.sync_copy(ov, out_hbm.at[pl.ds(nb + gg * _G, _G)])

            @pl.when(gg + 2 < ng)
            def _refill():
                start(gg + 2, slot)


def _edge_reduce(i1, i0, a_tab, bt_tab):
    mesh = plsc.VectorSubcoreMesh(core_axis_name="c", subcore_axis_name="s",
                                  num_cores=_NC, num_subcores=_NS)
    f = pl.kernel(
        _sc_body,
        out_type=jax.ShapeDtypeStruct((_N_OUT, _CW), jnp.int32),
        mesh=mesh,
        compiler_params=pltpu.CompilerParams(needs_layout_passes=False,
                                             use_tc_tiling_on_sc=False),
        scratch_types=[
            pltpu.VMEM((_NG0, _GK), jnp.int32),
            pltpu.VMEM((_NG0, _GK), jnp.int32),
            pltpu.VMEM((2, _GK, _CW), jnp.int32),
            pltpu.VMEM((2, _GK, _CW), jnp.int32),
            pltpu.VMEM((_G, _CW), jnp.int32),
            pltpu.VMEM_SHARED((_N, _CW), jnp.int32),
            pltpu.SemaphoreType.DMA,
            pltpu.SemaphoreType.DMA,
            pltpu.SemaphoreType.DMA,
            pltpu.SemaphoreType.DMA,
        ],
    )
    return f(i1, i0, a_tab, bt_tab)


def kernel(x, edge_index, W, b):
    n = x.shape[2]
    xs = x[0, :, :, 0]
    idx = edge_index.reshape(2, n, _K).astype(jnp.int32)
    idx = jnp.pad(idx, ((0, 0), (0, _N_IDX - n), (0, 0)))
    idxg = idx.reshape(2, _NG_TOT, _GK)
    b2d = b.reshape(1, _CO)

    a_tab, bt_tab = _build_tables(xs, W, b2d)
    out = _edge_reduce(idxg[1], idxg[0], a_tab, bt_tab)
    v = lax.bitcast_convert_type(out[:n], jnp.bfloat16)
    res = v.transpose(2, 1, 0).reshape(_CO, n)
    return res.astype(jnp.float32)[None, :, :, None]
